# baseline (speedup 1.0000x reference)
import functools

import jax
import jax.numpy as jnp
from jax import lax
from jax.experimental import pallas as pl
from jax.experimental.pallas import tpu as pltpu
from jax.experimental.pallas import tpu_sc as plsc

N_NODES = 10000
N_EDGES = 320000
N_REL = 8
D = 128
DH = 64

NC = 2
NS = 16
CHUNK = 128
CHUNKS_PER_TILE = 160
E_PAD = NS * CHUNKS_PER_TILE * CHUNK
STRIPE_OFF = 624


def _xw_body(f_ref, w_ref, o_ref):
    o_ref[0, 0] = jnp.dot(f_ref[...], w_ref[0, 0],
                          preferred_element_type=jnp.float32
                          ).astype(jnp.bfloat16)


def _compute_xw(feature, weight):
    bn = 2000
    nblk = N_NODES // bn
    wsplit = weight.reshape(N_REL, D, NC, DH).transpose(2, 0, 1, 3)
    return pl.pallas_call(
        _xw_body,
        grid=(nblk, N_REL, NC),
        in_specs=[
            pl.BlockSpec((bn, D), lambda b, r, c: (b, 0)),
            pl.BlockSpec((1, 1, D, DH), lambda b, r, c: (c, r, 0, 0)),
        ],
        out_specs=pl.BlockSpec((1, 1, bn, DH), lambda b, r, c: (c, r, b, 0)),
        out_shape=jax.ShapeDtypeStruct((NC, N_REL, N_NODES, DH), jnp.bfloat16),
    )(feature, wsplit)


NBUF = 4


NMETA = 8


def _sc_edge_kernel(xw_hbm, meta_hbm, norm_hbm, bias_hbm, out_hbm,
                    meta_v, rows_bf, rows_v, norm_v, bias_v, h_sh,
                    gsem, msem, ssem):
    c = lax.axis_index("c")
    s = lax.axis_index("s")
    off = s * CHUNKS_PER_TILE
    n_chunks = CHUNKS_PER_TILE
    pltpu.sync_copy(norm_hbm.at[pl.ds(off, CHUNKS_PER_TILE)], norm_v)
    pltpu.sync_copy(bias_hbm.at[0, pl.ds(c * DH, DH)], bias_v)

    def issue_meta(k, m):
        pltpu.async_copy(meta_hbm.at[off + k], meta_v.at[m], msem.at[m])

    def wait_meta(k, m):
        pltpu.make_async_copy(meta_hbm.at[off + k], meta_v.at[m],
                              msem.at[m]).wait()

    def compute_g(m):
        for j in range(8):
            sl = pl.ds(j * 16, 16)
            meta_v[m, 0, sl] = meta_v[m, 1, sl] * N_NODES + meta_v[m, 0, sl]

    def issue_gather(m, b):
        pltpu.async_copy(xw_hbm.at[c].at[meta_v.at[m, 0]], rows_bf.at[b],
                         gsem.at[b])

    def wait_gather(m, b):
        pltpu.make_async_copy(xw_hbm.at[c].at[meta_v.at[m, 0]], rows_bf.at[b],
                              gsem.at[b]).wait()

    zero16 = jnp.zeros((16,), jnp.float32)

    def _zrow(k, _):
        for j in range(DH // 16):
            rows_v[0, k, pl.ds(j * 16, 16)] = zero16
        return _
    lax.fori_loop(0, CHUNK, _zrow, None)
    for t in range(5):
        pltpu.sync_copy(rows_v.at[0],
                        h_sh.at[pl.ds(s * STRIPE_OFF + t * CHUNK, CHUNK)])
    plsc.subcore_barrier()

    def issue_scatter(m, b):
        pltpu.async_copy(rows_v.at[b], h_sh.at[meta_v.at[m, 2]], ssem.at[b],
                         add=True)

    def wait_scatter(m, b):
        pltpu.make_async_copy(rows_v.at[b], h_sh.at[meta_v.at[m, 2]],
                              ssem.at[b]).wait()

    for k in range(3):
        issue_meta(k, k)
    for k in range(2):
        wait_meta(k, k)
        compute_g(k)
        issue_gather(k, k)

    def _outer(o, _):
        for i in range(NMETA):
            k = o * NMETA + i
            b = i % NBUF
            m = i % NMETA
            pm = (i + 3) % NMETA
            gm = (i + 2) % NMETA
            pg = (i + 2) % NBUF
            wait_gather(m, b)

            @pl.when(k < n_chunks - 3)
            def _meta(k=k, pm=pm):
                issue_meta(k + 3, pm)

            @pl.when(k < n_chunks - 2)
            def _gather(k=k, gm=gm, pg=pg):
                @pl.when(k >= 2)
                def _wait_sc(gm=gm, pg=pg):
                    wait_scatter((gm + NBUF) % NMETA, pg)
                wait_meta(k + 2, gm)
                compute_g(gm)
                issue_gather(gm, pg)

            def _scale(g, _c, k=k, b=b):
                nv = norm_v[k, pl.ds(g * 16, 16)]
                for l in range(16):
                    nrm = nv[l]
                    e = g * 16 + l
                    for j in range(DH // 32):
                        vb = rows_bf[b, e, pl.ds(j * 32, 32)]
                        vf = vb.astype(jnp.float32)
                        sl = pl.ds(j * 32, 32)
                        rows_v[b, e, sl] = vf * nrm
                return _c
            lax.fori_loop(0, CHUNK // 16, _scale, None)
            issue_scatter(m, b)
        return _
    lax.fori_loop(0, n_chunks // NMETA, _outer, None)

    for i in range(NBUF):
        k = n_chunks - NBUF + i
        wait_scatter(k % NMETA, k % NBUF)

    plsc.subcore_barrier()
    for t in range(5):
        roff = s * STRIPE_OFF + t * CHUNK
        pltpu.sync_copy(h_sh.at[pl.ds(roff, CHUNK)], rows_v.at[0])

        def _fin(rr, _):
            for j in range(DH // 16):
                sl = pl.ds(j * 16, 16)
                x = rows_v[0, rr, sl] + bias_v[sl]
                e2x = jnp.exp(x + x)
                rows_v[0, rr, sl] = 1.0 - 2.0 / (e2x + 1.0)
            return _
        lax.fori_loop(0, CHUNK, _fin, None)
        pltpu.sync_copy(rows_v.at[0],
                        out_hbm.at[pl.ds(roff, CHUNK), pl.ds(c * DH, DH)])


def _sc_edge_sum(xw, meta, norm2, bias):
    mesh = plsc.VectorSubcoreMesh(core_axis_name="c", subcore_axis_name="s")
    k = functools.partial(
        pl.kernel,
        mesh=mesh,
        compiler_params=pltpu.CompilerParams(use_tc_tiling_on_sc=False),
        out_type=jax.ShapeDtypeStruct((N_NODES, D), jnp.float32),
        scratch_types=[
            pltpu.VMEM((NMETA, 4, CHUNK), jnp.int32),
            pltpu.VMEM((NBUF, CHUNK, DH), jnp.bfloat16),
            pltpu.VMEM((NBUF, CHUNK, DH), jnp.float32),
            pltpu.VMEM((CHUNKS_PER_TILE, CHUNK), jnp.float32),
            pltpu.VMEM((DH,), jnp.float32),
            pltpu.VMEM_SHARED((N_NODES, DH), jnp.float32),
            pltpu.SemaphoreType.DMA((NBUF,)),
            pltpu.SemaphoreType.DMA((NMETA,)),
            pltpu.SemaphoreType.DMA((NBUF,)),
        ],
    )(_sc_edge_kernel)
    return k(xw, meta, norm2, bias)


def kernel(feature, edge_index, rel_type, norm, weight, bias):
    xw = _compute_xw(feature, weight).reshape(NC, N_REL * N_NODES, DH)

    pad = E_PAD - N_EDGES
    src2 = jnp.pad(edge_index[0], (0, pad)).reshape(-1, CHUNK)
    dst2 = jnp.pad(edge_index[1], (0, pad)).reshape(-1, CHUNK)
    rel2 = jnp.pad(rel_type, (0, pad)).reshape(-1, CHUNK)
    norm2 = jnp.pad(norm, (0, pad)).reshape(-1, CHUNK)
    meta = jnp.stack([src2, rel2, dst2, jnp.zeros_like(src2)], axis=1)

    return _sc_edge_sum(xw, meta, norm2, bias)


# speedup vs baseline: 1.0830x; 1.0830x over previous
import functools

import jax
import jax.numpy as jnp
from jax import lax
from jax.experimental import pallas as pl
from jax.experimental.pallas import tpu as pltpu
from jax.experimental.pallas import tpu_sc as plsc

N_NODES = 10000
N_EDGES = 320000
N_REL = 8
D = 128
DH = 64

NC = 2
NS = 16
CHUNK = 128
CHUNKS_PER_TILE = 160
E_PAD = NS * CHUNKS_PER_TILE * CHUNK
STRIPE_OFF = 624


def _xw_body(f_ref, w_ref, o_ref):
    o_ref[0, 0] = jnp.dot(f_ref[...], w_ref[0, 0],
                          preferred_element_type=jnp.float32
                          ).astype(jnp.bfloat16)


def _compute_xw(feature, weight):
    bn = 2000
    nblk = N_NODES // bn
    wsplit = weight.reshape(N_REL, D, NC, DH).transpose(2, 0, 1, 3)
    return pl.pallas_call(
        _xw_body,
        grid=(nblk, N_REL, NC),
        in_specs=[
            pl.BlockSpec((bn, D), lambda b, r, c: (b, 0)),
            pl.BlockSpec((1, 1, D, DH), lambda b, r, c: (c, r, 0, 0)),
        ],
        out_specs=pl.BlockSpec((1, 1, bn, DH), lambda b, r, c: (c, r, b, 0)),
        out_shape=jax.ShapeDtypeStruct((NC, N_REL, N_NODES, DH), jnp.bfloat16),
    )(feature, wsplit)


NBUF = 4


NMETA = 8


def _sc_edge_kernel(xw_hbm, meta_hbm, norm_hbm, out_hbm,
                    meta_v, rows_bf, rows_v, norm_v, h_sh,
                    gsem, msem, ssem):
    c = lax.axis_index("c")
    s = lax.axis_index("s")
    off = s * CHUNKS_PER_TILE
    n_chunks = CHUNKS_PER_TILE
    pltpu.sync_copy(norm_hbm.at[pl.ds(off, CHUNKS_PER_TILE)], norm_v)

    def issue_meta(k, m):
        pltpu.async_copy(meta_hbm.at[off + k], meta_v.at[m], msem.at[m])

    def wait_meta(k, m):
        pltpu.make_async_copy(meta_hbm.at[off + k], meta_v.at[m],
                              msem.at[m]).wait()

    def compute_g(m):
        for j in range(8):
            sl = pl.ds(j * 16, 16)
            meta_v[m, 0, sl] = meta_v[m, 1, sl] * N_NODES + meta_v[m, 0, sl]

    def issue_gather(m, b):
        pltpu.async_copy(xw_hbm.at[c].at[meta_v.at[m, 0]], rows_bf.at[b],
                         gsem.at[b])

    def wait_gather(m, b):
        pltpu.make_async_copy(xw_hbm.at[c].at[meta_v.at[m, 0]], rows_bf.at[b],
                              gsem.at[b]).wait()

    zero16 = jnp.zeros((16,), jnp.float32)

    def _zrow(k, _):
        for j in range(DH // 16):
            rows_v[0, k, pl.ds(j * 16, 16)] = zero16
        return _
    lax.fori_loop(0, CHUNK, _zrow, None)
    for t in range(5):
        pltpu.sync_copy(rows_v.at[0],
                        h_sh.at[pl.ds(s * STRIPE_OFF + t * CHUNK, CHUNK)])
    plsc.subcore_barrier()

    def issue_scatter(m, b):
        pltpu.async_copy(rows_v.at[b], h_sh.at[meta_v.at[m, 2]], ssem.at[b],
                         add=True)

    def wait_scatter(m, b):
        pltpu.make_async_copy(rows_v.at[b], h_sh.at[meta_v.at[m, 2]],
                              ssem.at[b]).wait()

    for k in range(3):
        issue_meta(k, k)
    for k in range(2):
        wait_meta(k, k)
        compute_g(k)
        issue_gather(k, k)

    def _outer(o, _):
        for i in range(NMETA):
            k = o * NMETA + i
            b = i % NBUF
            m = i % NMETA
            pm = (i + 3) % NMETA
            gm = (i + 2) % NMETA
            pg = (i + 2) % NBUF
            wait_gather(m, b)

            @pl.when(k < n_chunks - 3)
            def _meta(k=k, pm=pm):
                issue_meta(k + 3, pm)

            @pl.when(k < n_chunks - 2)
            def _gather(k=k, gm=gm, pg=pg):
                @pl.when(k >= 2)
                def _wait_sc(gm=gm, pg=pg):
                    wait_scatter((gm + NBUF) % NMETA, pg)
                wait_meta(k + 2, gm)
                compute_g(gm)
                issue_gather(gm, pg)

            def _scale(g, _c, k=k, b=b):
                nv = norm_v[k, pl.ds(g * 16, 16)]
                for l in range(16):
                    nrm = nv[l]
                    e = g * 16 + l
                    for j in range(DH // 32):
                        vb = rows_bf[b, e, pl.ds(j * 32, 32)]
                        vf = vb.astype(jnp.float32)
                        sl = pl.ds(j * 32, 32)
                        rows_v[b, e, sl] = vf * nrm
                return _c
            lax.fori_loop(0, CHUNK // 16, _scale, None)
            issue_scatter(m, b)
        return _
    lax.fori_loop(0, n_chunks // NMETA, _outer, None)

    for i in range(NBUF):
        k = n_chunks - NBUF + i
        wait_scatter(k % NMETA, k % NBUF)

    plsc.subcore_barrier()
    for t in range(5):
        roff = s * STRIPE_OFF + t * CHUNK
        pltpu.sync_copy(h_sh.at[pl.ds(roff, CHUNK)],
                        out_hbm.at[c, pl.ds(roff, CHUNK)])


def _sc_edge_sum(xw, meta, norm2):
    mesh = plsc.VectorSubcoreMesh(core_axis_name="c", subcore_axis_name="s")
    k = functools.partial(
        pl.kernel,
        mesh=mesh,
        compiler_params=pltpu.CompilerParams(use_tc_tiling_on_sc=False),
        out_type=jax.ShapeDtypeStruct((NC, N_NODES, DH), jnp.float32),
        scratch_types=[
            pltpu.VMEM((NMETA, 4, CHUNK), jnp.int32),
            pltpu.VMEM((NBUF, CHUNK, DH), jnp.bfloat16),
            pltpu.VMEM((NBUF, CHUNK, DH), jnp.float32),
            pltpu.VMEM((CHUNKS_PER_TILE, CHUNK), jnp.float32),
            pltpu.VMEM_SHARED((N_NODES, DH), jnp.float32),
            pltpu.SemaphoreType.DMA((NBUF,)),
            pltpu.SemaphoreType.DMA((NMETA,)),
            pltpu.SemaphoreType.DMA((NBUF,)),
        ],
    )(_sc_edge_kernel)
    return k(xw, meta, norm2)


def _fin_body(p_ref, b_ref, o_ref):
    h = jnp.concatenate([p_ref[0], p_ref[1]], axis=-1)
    o_ref[...] = jnp.tanh(h + b_ref[...])


def _finish(partials, bias):
    bn = 2000
    nblk = N_NODES // bn
    return pl.pallas_call(
        _fin_body,
        grid=(nblk,),
        in_specs=[
            pl.BlockSpec((NC, bn, DH), lambda b: (0, b, 0)),
            pl.BlockSpec((1, D), lambda b: (0, 0)),
        ],
        out_specs=pl.BlockSpec((bn, D), lambda b: (b, 0)),
        out_shape=jax.ShapeDtypeStruct((N_NODES, D), jnp.float32),
    )(partials, bias)


def kernel(feature, edge_index, rel_type, norm, weight, bias):
    xw = _compute_xw(feature, weight).reshape(NC, N_REL * N_NODES, DH)

    pad = E_PAD - N_EDGES
    src2 = jnp.pad(edge_index[0], (0, pad)).reshape(-1, CHUNK)
    dst2 = jnp.pad(edge_index[1], (0, pad)).reshape(-1, CHUNK)
    rel2 = jnp.pad(rel_type, (0, pad)).reshape(-1, CHUNK)
    norm2 = jnp.pad(norm, (0, pad)).reshape(-1, CHUNK)
    meta = jnp.stack([src2, rel2, dst2, jnp.zeros_like(src2)], axis=1)

    partials = _sc_edge_sum(xw, meta, norm2)
    return _finish(partials, bias)


# speedup vs baseline: 1.1623x; 1.0732x over previous
import functools

import jax
import jax.numpy as jnp
from jax import lax
from jax.experimental import pallas as pl
from jax.experimental.pallas import tpu as pltpu
from jax.experimental.pallas import tpu_sc as plsc

N_NODES = 10000
N_EDGES = 320000
N_REL = 8
D = 128
DH = 64

NC = 2
NS = 16
CHUNK = 128
CHUNKS_PER_TILE = 160
E_PAD = NS * CHUNKS_PER_TILE * CHUNK
STRIPE_OFF = 624


def _xw_body(f_ref, w_ref, o_ref):
    res = jnp.dot(f_ref[...], w_ref[0],
                  preferred_element_type=jnp.float32).astype(jnp.bfloat16)
    o_ref[0, 0] = res[:, 0 * DH:1 * DH]
    o_ref[1, 0] = res[:, 1 * DH:2 * DH]
    o_ref[0, 1] = res[:, 2 * DH:3 * DH]
    o_ref[1, 1] = res[:, 3 * DH:4 * DH]


def _compute_xw(feature, weight):
    bn = 2000
    nblk = N_NODES // bn
    wpair = weight.reshape(N_REL // 2, 2, D, D).transpose(0, 2, 1, 3)
    wpair = wpair.reshape(N_REL // 2, D, 2 * D)
    return pl.pallas_call(
        _xw_body,
        grid=(nblk, N_REL // 2),
        in_specs=[
            pl.BlockSpec((bn, D), lambda b, rp: (b, 0)),
            pl.BlockSpec((1, D, 2 * D), lambda b, rp: (rp, 0, 0)),
        ],
        out_specs=pl.BlockSpec((NC, 2, bn, DH), lambda b, rp: (0, rp, b, 0)),
        out_shape=jax.ShapeDtypeStruct((NC, N_REL, N_NODES, DH), jnp.bfloat16),
    )(feature, wpair)


NBUF = 4


NMETA = 8


def _sc_edge_kernel(xw_hbm, meta_hbm, norm_hbm, out_hbm,
                    meta_v, rows_bf, rows_v, norm_v, h_sh,
                    gsem, msem, ssem):
    c = lax.axis_index("c")
    s = lax.axis_index("s")
    off = s * CHUNKS_PER_TILE
    n_chunks = CHUNKS_PER_TILE
    pltpu.sync_copy(norm_hbm.at[pl.ds(off, CHUNKS_PER_TILE)], norm_v)

    def issue_meta(k, m):
        pltpu.async_copy(meta_hbm.at[off + k], meta_v.at[m], msem.at[m])

    def wait_meta(k, m):
        pltpu.make_async_copy(meta_hbm.at[off + k], meta_v.at[m],
                              msem.at[m]).wait()

    def compute_g(m):
        for j in range(8):
            sl = pl.ds(j * 16, 16)
            meta_v[m, 0, sl] = meta_v[m, 1, sl] * N_NODES + meta_v[m, 0, sl]

    def issue_gather(m, b):
        pltpu.async_copy(xw_hbm.at[c].at[meta_v.at[m, 0]], rows_bf.at[b],
                         gsem.at[b])

    def wait_gather(m, b):
        pltpu.make_async_copy(xw_hbm.at[c].at[meta_v.at[m, 0]], rows_bf.at[b],
                              gsem.at[b]).wait()

    zero16 = jnp.zeros((16,), jnp.float32)

    def _zrow(k, _):
        for j in range(DH // 16):
            rows_v[0, k, pl.ds(j * 16, 16)] = zero16
        return _
    lax.fori_loop(0, CHUNK, _zrow, None)
    for t in range(5):
        pltpu.sync_copy(rows_v.at[0],
                        h_sh.at[pl.ds(s * STRIPE_OFF + t * CHUNK, CHUNK)])
    plsc.subcore_barrier()

    def issue_scatter(m, b):
        pltpu.async_copy(rows_v.at[b], h_sh.at[meta_v.at[m, 2]], ssem.at[b],
                         add=True)

    def wait_scatter(m, b):
        pltpu.make_async_copy(rows_v.at[b], h_sh.at[meta_v.at[m, 2]],
                              ssem.at[b]).wait()

    for k in range(3):
        issue_meta(k, k)
    for k in range(2):
        wait_meta(k, k)
        compute_g(k)
        issue_gather(k, k)

    def _outer(o, _):
        for i in range(NMETA):
            k = o * NMETA + i
            b = i % NBUF
            m = i % NMETA
            pm = (i + 3) % NMETA
            gm = (i + 2) % NMETA
            pg = (i + 2) % NBUF
            wait_gather(m, b)

            @pl.when(k < n_chunks - 3)
            def _meta(k=k, pm=pm):
                issue_meta(k + 3, pm)

            @pl.when(k < n_chunks - 2)
            def _gather(k=k, gm=gm, pg=pg):
                @pl.when(k >= 2)
                def _wait_sc(gm=gm, pg=pg):
                    wait_scatter((gm + NBUF) % NMETA, pg)
                wait_meta(k + 2, gm)
                compute_g(gm)
                issue_gather(gm, pg)

            def _scale(g, _c, k=k, b=b):
                nv = norm_v[k, pl.ds(g * 16, 16)]
                for l in range(16):
                    nrm = nv[l]
                    e = g * 16 + l
                    for j in range(DH // 32):
                        vb = rows_bf[b, e, pl.ds(j * 32, 32)]
                        vf = vb.astype(jnp.float32)
                        sl = pl.ds(j * 32, 32)
                        rows_v[b, e, sl] = vf * nrm
                return _c
            lax.fori_loop(0, CHUNK // 16, _scale, None)
            issue_scatter(m, b)
        return _
    lax.fori_loop(0, n_chunks // NMETA, _outer, None)

    for i in range(NBUF):
        k = n_chunks - NBUF + i
        wait_scatter(k % NMETA, k % NBUF)

    plsc.subcore_barrier()
    for t in range(5):
        roff = s * STRIPE_OFF + t * CHUNK
        pltpu.sync_copy(h_sh.at[pl.ds(roff, CHUNK)],
                        out_hbm.at[c, pl.ds(roff, CHUNK)])


def _sc_edge_sum(xw, meta, norm2):
    mesh = plsc.VectorSubcoreMesh(core_axis_name="c", subcore_axis_name="s")
    k = functools.partial(
        pl.kernel,
        mesh=mesh,
        compiler_params=pltpu.CompilerParams(use_tc_tiling_on_sc=False),
        out_type=jax.ShapeDtypeStruct((NC, N_NODES, DH), jnp.float32),
        scratch_types=[
            pltpu.VMEM((NMETA, 4, CHUNK), jnp.int32),
            pltpu.VMEM((NBUF, CHUNK, DH), jnp.bfloat16),
            pltpu.VMEM((NBUF, CHUNK, DH), jnp.float32),
            pltpu.VMEM((CHUNKS_PER_TILE, CHUNK), jnp.float32),
            pltpu.VMEM_SHARED((N_NODES, DH), jnp.float32),
            pltpu.SemaphoreType.DMA((NBUF,)),
            pltpu.SemaphoreType.DMA((NMETA,)),
            pltpu.SemaphoreType.DMA((NBUF,)),
        ],
    )(_sc_edge_kernel)
    return k(xw, meta, norm2)


def _fin_body(p_ref, b_ref, o_ref):
    h = jnp.concatenate([p_ref[0], p_ref[1]], axis=-1)
    o_ref[...] = jnp.tanh(h + b_ref[...])


def _finish(partials, bias):
    bn = 2000
    nblk = N_NODES // bn
    return pl.pallas_call(
        _fin_body,
        grid=(nblk,),
        in_specs=[
            pl.BlockSpec((NC, bn, DH), lambda b: (0, b, 0)),
            pl.BlockSpec((1, D), lambda b: (0, 0)),
        ],
        out_specs=pl.BlockSpec((bn, D), lambda b: (b, 0)),
        out_shape=jax.ShapeDtypeStruct((N_NODES, D), jnp.float32),
    )(partials, bias)


def kernel(feature, edge_index, rel_type, norm, weight, bias):
    xw = _compute_xw(feature, weight).reshape(NC, N_REL * N_NODES, DH)

    pad = E_PAD - N_EDGES
    src2 = jnp.pad(edge_index[0], (0, pad)).reshape(-1, CHUNK)
    dst2 = jnp.pad(edge_index[1], (0, pad)).reshape(-1, CHUNK)
    rel2 = jnp.pad(rel_type, (0, pad)).reshape(-1, CHUNK)
    norm2 = jnp.pad(norm, (0, pad)).reshape(-1, CHUNK)
    meta = jnp.stack([src2, rel2, dst2, jnp.zeros_like(src2)], axis=1)

    partials = _sc_edge_sum(xw, meta, norm2)
    return _finish(partials, bias)
